# baseline (device time: 28426 ns/iter reference)
import jax
import jax.numpy as jnp
from jax import lax
from jax.experimental import pallas as pl
from jax.experimental.pallas import tpu as pltpu

N_DEV = 4
NQ = 4


def kernel(x, W1, W2):
    m, _ = x.shape
    n = W2.shape[1]
    mc = m // N_DEV
    qr = mc // NQ
    hr = mc // 2

    def body(x_ref, w1_ref, w2_ref, out_ref,
             send_buf, rs_buf, bc_src, bc_buf,
             xb_ref, w1b_ref, w2b_ref,
             rs_send_sems, rs_recv_sems, bc_send_sems, bc_recv_sems):
        d = lax.axis_index("i")

        barrier_sem = pltpu.get_barrier_semaphore()
        for kk in range(1, N_DEV):
            pl.semaphore_signal(
                barrier_sem, inc=1,
                device_id=((d + kk) % N_DEV,),
                device_id_type=pl.DeviceIdType.MESH,
            )
        pl.semaphore_wait(barrier_sem, N_DEV - 1)

        xb_ref[:, :] = x_ref[:, :].astype(jnp.bfloat16)
        w1b_ref[:, :] = w1_ref[:, :].astype(jnp.bfloat16)

        def mlp_rows(rows):
            h = jnp.maximum(
                jnp.dot(xb_ref[rows, :], w1b_ref[:, :],
                        preferred_element_type=jnp.float32),
                0.0,
            ).astype(jnp.bfloat16)
            return jnp.dot(h, w2b_ref[:, :], preferred_element_type=jnp.float32)

        sends = []

        def fire_quarters(kk, qs):
            c = (d + kk) % N_DEV
            slot = N_DEV - 1 - kk
            for q in qs:
                rows_q = pl.ds(q * qr, qr)
                rdma = pltpu.make_async_remote_copy(
                    src_ref=send_buf.at[kk - 1, rows_q],
                    dst_ref=rs_buf.at[slot, rows_q],
                    send_sem=rs_send_sems.at[kk - 1, q],
                    recv_sem=rs_recv_sems.at[slot, q],
                    device_id=(c,),
                    device_id_type=pl.DeviceIdType.MESH,
                )
                rdma.start()
                sends.append(rdma)

        c1 = (d + 1) % N_DEV
        h1_raw = jnp.dot(xb_ref[pl.ds(c1 * mc, mc), :], w1b_ref[:, :],
                         preferred_element_type=jnp.float32)
        w2b_ref[:, :] = w2_ref[:, :].astype(jnp.bfloat16)
        h1 = jnp.maximum(h1_raw, 0.0).astype(jnp.bfloat16)
        send_buf[0, :, :] = jnp.dot(
            h1, w2b_ref[:, :], preferred_element_type=jnp.float32
        ).astype(jnp.bfloat16)
        fire_quarters(1, range(NQ))

        c2 = (d + 2) % N_DEV
        send_buf[1, :, :] = mlp_rows(pl.ds(c2 * mc, mc)).astype(jnp.bfloat16)
        fire_quarters(2, range(NQ))

        c3 = (d + 3) % N_DEV
        for j in range(2):
            half = mlp_rows(pl.ds(c3 * mc + j * hr, hr)).astype(jnp.bfloat16)
            send_buf[2, pl.ds(j * hr, hr), :] = half
            fire_quarters(3, (2 * j, 2 * j + 1))

        def reduce_bc_quarter(q, own_q):
            rows_q = pl.ds(q * qr, qr)
            acc = own_q
            for slot in (2, 1, 0):
                recv = pltpu.make_async_remote_copy(
                    src_ref=rs_buf.at[slot, rows_q],
                    dst_ref=rs_buf.at[slot, rows_q],
                    send_sem=rs_send_sems.at[0, 0],
                    recv_sem=rs_recv_sems.at[slot, q],
                    device_id=(d,),
                    device_id_type=pl.DeviceIdType.MESH,
                )
                recv.wait_recv()
                acc = acc + rs_buf[slot, q * qr:(q + 1) * qr, :].astype(
                    jnp.float32)
            out_ref[pl.ds(d * mc + q * qr, qr), :] = acc
            bc_src[rows_q, :] = acc.astype(jnp.bfloat16)
            for kk in range(1, N_DEV):
                t = (d + kk) % N_DEV
                slot = N_DEV - 1 - kk
                rdma = pltpu.make_async_remote_copy(
                    src_ref=bc_src.at[rows_q],
                    dst_ref=bc_buf.at[slot, rows_q],
                    send_sem=bc_send_sems.at[kk - 1, q],
                    recv_sem=bc_recv_sems.at[slot, q],
                    device_id=(t,),
                    device_id_type=pl.DeviceIdType.MESH,
                )
                rdma.start()
                sends.append(rdma)

        for j in range(2):
            own_half = mlp_rows(pl.ds(d * mc + j * hr, hr))
            reduce_bc_quarter(2 * j, own_half[:qr, :])
            reduce_bc_quarter(2 * j + 1, own_half[qr:, :])

        for slot in range(N_DEV - 1):
            o = (d + slot + 1) % N_DEV
            for q in range(NQ):
                rows_q = pl.ds(q * qr, qr)
                recv = pltpu.make_async_remote_copy(
                    src_ref=bc_buf.at[slot, rows_q],
                    dst_ref=bc_buf.at[slot, rows_q],
                    send_sem=bc_send_sems.at[0, 0],
                    recv_sem=bc_recv_sems.at[slot, q],
                    device_id=(d,),
                    device_id_type=pl.DeviceIdType.MESH,
                )
                recv.wait_recv()
                out_ref[pl.ds(o * mc + q * qr, qr), :] = bc_buf[
                    slot, q * qr:(q + 1) * qr, :].astype(jnp.float32)

        for r in sends:
            r.wait_send()

    return pl.pallas_call(
        body,
        out_shape=jax.ShapeDtypeStruct((m, n), jnp.float32),
        in_specs=[
            pl.BlockSpec(memory_space=pltpu.VMEM),
            pl.BlockSpec(memory_space=pltpu.VMEM),
            pl.BlockSpec(memory_space=pltpu.VMEM),
        ],
        out_specs=pl.BlockSpec(memory_space=pltpu.VMEM),
        scratch_shapes=[
            pltpu.VMEM((N_DEV - 1, mc, n), jnp.bfloat16),
            pltpu.VMEM((N_DEV - 1, mc, n), jnp.bfloat16),
            pltpu.VMEM((mc, n), jnp.bfloat16),
            pltpu.VMEM((N_DEV - 1, mc, n), jnp.bfloat16),
            pltpu.VMEM(x.shape, jnp.bfloat16),
            pltpu.VMEM(W1.shape, jnp.bfloat16),
            pltpu.VMEM(W2.shape, jnp.bfloat16),
            pltpu.SemaphoreType.DMA((N_DEV - 1, NQ)),
            pltpu.SemaphoreType.DMA((N_DEV - 1, NQ)),
            pltpu.SemaphoreType.DMA((N_DEV - 1, NQ)),
            pltpu.SemaphoreType.DMA((N_DEV - 1, NQ)),
        ],
        compiler_params=pltpu.CompilerParams(collective_id=0),
    )(x, W1, W2)


# device time: 28301 ns/iter; 1.0044x vs baseline; 1.0044x over previous
import jax
import jax.numpy as jnp
from jax import lax
from jax.experimental import pallas as pl
from jax.experimental.pallas import tpu as pltpu

N_DEV = 4
NQ = 4


def kernel(x, W1, W2):
    m, _ = x.shape
    n = W2.shape[1]
    mc = m // N_DEV
    qr = mc // NQ
    hr = mc // 2

    def body(x_ref, w1_ref, w2_ref, out_ref,
             send_buf, rs_buf,
             xb_ref, w1b_ref, w2b_ref,
             rs_send_sems, rs_recv_sems, bc_send_sems, bc_recv_sems):
        d = lax.axis_index("i")

        barrier_sem = pltpu.get_barrier_semaphore()
        for kk in range(1, N_DEV):
            pl.semaphore_signal(
                barrier_sem, inc=1,
                device_id=((d + kk) % N_DEV,),
                device_id_type=pl.DeviceIdType.MESH,
            )
        pl.semaphore_wait(barrier_sem, N_DEV - 1)

        xb_ref[:, :] = x_ref[:, :].astype(jnp.bfloat16)
        w1b_ref[:, :] = w1_ref[:, :].astype(jnp.bfloat16)

        def mlp_rows(rows):
            h = jnp.maximum(
                jnp.dot(xb_ref[rows, :], w1b_ref[:, :],
                        preferred_element_type=jnp.float32),
                0.0,
            ).astype(jnp.bfloat16)
            return jnp.dot(h, w2b_ref[:, :], preferred_element_type=jnp.float32)

        sends = []

        def fire_quarters(kk, qs):
            c = (d + kk) % N_DEV
            slot = N_DEV - 1 - kk
            for q in qs:
                rows_q = pl.ds(q * qr, qr)
                rdma = pltpu.make_async_remote_copy(
                    src_ref=send_buf.at[kk - 1, rows_q],
                    dst_ref=rs_buf.at[slot, rows_q],
                    send_sem=rs_send_sems.at[kk - 1, q],
                    recv_sem=rs_recv_sems.at[slot, q],
                    device_id=(c,),
                    device_id_type=pl.DeviceIdType.MESH,
                )
                rdma.start()
                sends.append(rdma)

        c1 = (d + 1) % N_DEV
        h1_raw = jnp.dot(xb_ref[pl.ds(c1 * mc, mc), :], w1b_ref[:, :],
                         preferred_element_type=jnp.float32)
        w2b_ref[:, :] = w2_ref[:, :].astype(jnp.bfloat16)
        h1 = jnp.maximum(h1_raw, 0.0).astype(jnp.bfloat16)
        send_buf[0, :, :] = jnp.dot(
            h1, w2b_ref[:, :], preferred_element_type=jnp.float32
        ).astype(jnp.bfloat16)
        fire_quarters(1, range(NQ))

        c2 = (d + 2) % N_DEV
        send_buf[1, :, :] = mlp_rows(pl.ds(c2 * mc, mc)).astype(jnp.bfloat16)
        fire_quarters(2, range(NQ))

        c3 = (d + 3) % N_DEV
        for j in range(2):
            half = mlp_rows(pl.ds(c3 * mc + j * hr, hr)).astype(jnp.bfloat16)
            send_buf[2, pl.ds(j * hr, hr), :] = half
            fire_quarters(3, (2 * j, 2 * j + 1))

        def reduce_bc_quarter(q, own_q):
            out_rows = pl.ds(d * mc + q * qr, qr)
            acc = own_q
            for slot in (2, 1, 0):
                recv = pltpu.make_async_remote_copy(
                    src_ref=rs_buf.at[slot, pl.ds(q * qr, qr)],
                    dst_ref=rs_buf.at[slot, pl.ds(q * qr, qr)],
                    send_sem=rs_send_sems.at[0, 0],
                    recv_sem=rs_recv_sems.at[slot, q],
                    device_id=(d,),
                    device_id_type=pl.DeviceIdType.MESH,
                )
                recv.wait_recv()
                acc = acc + rs_buf[slot, q * qr:(q + 1) * qr, :].astype(
                    jnp.float32)
            out_ref[out_rows, :] = acc.astype(jnp.bfloat16)
            for kk in range(1, N_DEV):
                t = (d + kk) % N_DEV
                slot = N_DEV - 1 - kk
                rdma = pltpu.make_async_remote_copy(
                    src_ref=out_ref.at[out_rows],
                    dst_ref=out_ref.at[out_rows],
                    send_sem=bc_send_sems.at[kk - 1, q],
                    recv_sem=bc_recv_sems.at[slot, q],
                    device_id=(t,),
                    device_id_type=pl.DeviceIdType.MESH,
                )
                rdma.start()
                sends.append(rdma)

        for j in range(2):
            own_half = mlp_rows(pl.ds(d * mc + j * hr, hr))
            reduce_bc_quarter(2 * j, own_half[:qr, :])
            reduce_bc_quarter(2 * j + 1, own_half[qr:, :])

        for slot in range(N_DEV - 1):
            o = (d + slot + 1) % N_DEV
            for q in range(NQ):
                rows_o = pl.ds(o * mc + q * qr, qr)
                recv = pltpu.make_async_remote_copy(
                    src_ref=out_ref.at[rows_o],
                    dst_ref=out_ref.at[rows_o],
                    send_sem=bc_send_sems.at[0, 0],
                    recv_sem=bc_recv_sems.at[slot, q],
                    device_id=(d,),
                    device_id_type=pl.DeviceIdType.MESH,
                )
                recv.wait_recv()

        for r in sends:
            r.wait_send()

    return pl.pallas_call(
        body,
        out_shape=jax.ShapeDtypeStruct((m, n), jnp.bfloat16),
        in_specs=[
            pl.BlockSpec(memory_space=pltpu.VMEM),
            pl.BlockSpec(memory_space=pltpu.VMEM),
            pl.BlockSpec(memory_space=pltpu.VMEM),
        ],
        out_specs=pl.BlockSpec(memory_space=pltpu.VMEM),
        scratch_shapes=[
            pltpu.VMEM((N_DEV - 1, mc, n), jnp.bfloat16),
            pltpu.VMEM((N_DEV - 1, mc, n), jnp.bfloat16),
            pltpu.VMEM(x.shape, jnp.bfloat16),
            pltpu.VMEM(W1.shape, jnp.bfloat16),
            pltpu.VMEM(W2.shape, jnp.bfloat16),
            pltpu.SemaphoreType.DMA((N_DEV - 1, NQ)),
            pltpu.SemaphoreType.DMA((N_DEV - 1, NQ)),
            pltpu.SemaphoreType.DMA((N_DEV - 1, NQ)),
            pltpu.SemaphoreType.DMA((N_DEV - 1, NQ)),
        ],
        compiler_params=pltpu.CompilerParams(collective_id=0),
    )(x, W1, W2)


# device time: 27999 ns/iter; 1.0153x vs baseline; 1.0108x over previous
import jax
import jax.numpy as jnp
from jax import lax
from jax.experimental import pallas as pl
from jax.experimental.pallas import tpu as pltpu

N_DEV = 4


def kernel(x, W1, W2):
    m, _ = x.shape
    n = W2.shape[1]
    mc = m // N_DEV
    hr = mc // 2

    def body(x_ref, w1_ref, w2_ref, out_ref,
             send_buf, rs_buf, relay_buf, comb_buf,
             xb_ref, w1b_ref, w2b_ref,
             rs_send_sems, rs_recv_sems, bc_send_sems, bc_recv_sems):
        d = lax.axis_index("i")
        right = (d + 1) % N_DEV
        left = (d - 1) % N_DEV

        def rows_half(c, j):
            return pl.ds(c * mc + j * hr, hr)

        barrier_sem = pltpu.get_barrier_semaphore()
        for nbr in (left, right):
            pl.semaphore_signal(
                barrier_sem, inc=1,
                device_id=(nbr,), device_id_type=pl.DeviceIdType.MESH,
            )
        pl.semaphore_wait(barrier_sem, 2)

        xb_ref[:, :] = x_ref[:, :].astype(jnp.bfloat16)
        w1b_ref[:, :] = w1_ref[:, :].astype(jnp.bfloat16)

        sends = []

        def rdma_to(src, dst, s_sem, r_sem, target):
            r = pltpu.make_async_remote_copy(
                src_ref=src, dst_ref=dst, send_sem=s_sem, recv_sem=r_sem,
                device_id=(target,), device_id_type=pl.DeviceIdType.MESH,
            )
            r.start()
            sends.append(r)

        def wait_recv_on(r_sem, ref):
            r = pltpu.make_async_remote_copy(
                src_ref=ref, dst_ref=ref,
                send_sem=rs_send_sems.at[0], recv_sem=r_sem,
                device_id=(d,), device_id_type=pl.DeviceIdType.MESH,
            )
            r.wait_recv()

        c2 = (d + 2) % N_DEV
        h2_raw = jnp.dot(xb_ref[pl.ds(c2 * mc, mc), :], w1b_ref[:, :],
                         preferred_element_type=jnp.float32)
        w2b_ref[:, :] = w2_ref[:, :].astype(jnp.bfloat16)
        h2 = jnp.maximum(h2_raw, 0.0).astype(jnp.bfloat16)
        send_buf[0, :, :] = jnp.dot(
            h2, w2b_ref[:, :], preferred_element_type=jnp.float32
        ).astype(jnp.bfloat16)
        rdma_to(send_buf.at[0, pl.ds(0, hr)], relay_buf.at[0],
                rs_send_sems.at[0], rs_recv_sems.at[4], left)
        rdma_to(send_buf.at[0, pl.ds(hr, hr)], relay_buf.at[1],
                rs_send_sems.at[1], rs_recv_sems.at[5], right)

        def mlp_rows(rows):
            h = jnp.maximum(
                jnp.dot(xb_ref[rows, :], w1b_ref[:, :],
                        preferred_element_type=jnp.float32),
                0.0,
            ).astype(jnp.bfloat16)
            return jnp.dot(h, w2b_ref[:, :], preferred_element_type=jnp.float32)

        c1 = right
        send_buf[1, :, :] = mlp_rows(pl.ds(c1 * mc, mc)).astype(jnp.bfloat16)
        rdma_to(send_buf.at[1, pl.ds(0, hr)], rs_buf.at[0],
                rs_send_sems.at[2], rs_recv_sems.at[0], right)

        c3 = left
        send_buf[2, :, :] = mlp_rows(pl.ds(c3 * mc, mc)).astype(jnp.bfloat16)
        rdma_to(send_buf.at[2, pl.ds(hr, hr)], rs_buf.at[2],
                rs_send_sems.at[4], rs_recv_sems.at[2], left)

        wait_recv_on(rs_recv_sems.at[5], relay_buf.at[1])
        comb_buf[0, :, :] = send_buf[1, hr:, :] + relay_buf[1, :, :]
        rdma_to(comb_buf.at[0], rs_buf.at[3],
                rs_send_sems.at[3], rs_recv_sems.at[3], right)
        wait_recv_on(rs_recv_sems.at[4], relay_buf.at[0])
        comb_buf[1, :, :] = send_buf[2, :hr, :] + relay_buf[0, :, :]
        rdma_to(comb_buf.at[1], rs_buf.at[1],
                rs_send_sems.at[5], rs_recv_sems.at[1], left)

        own = mlp_rows(pl.ds(d * mc, mc))
        for j, (s1, s2) in enumerate(((0, 1), (2, 3))):
            wait_recv_on(rs_recv_sems.at[s1], rs_buf.at[s1])
            wait_recv_on(rs_recv_sems.at[s2], rs_buf.at[s2])
            acc = (own[j * hr:(j + 1) * hr, :]
                   + rs_buf[s1, :, :].astype(jnp.float32)
                   + rs_buf[s2, :, :].astype(jnp.float32))
            out_ref[rows_half(d, j), :] = acc.astype(jnp.bfloat16)
            rdma_to(out_ref.at[rows_half(d, j)], out_ref.at[rows_half(d, j)],
                    bc_send_sems.at[j], bc_recv_sems.at[2 + j], right)
            rdma_to(out_ref.at[rows_half(d, j)], out_ref.at[rows_half(d, j)],
                    bc_send_sems.at[2 + j], bc_recv_sems.at[j], left)

        wait_recv_on(bc_recv_sems.at[1], out_ref.at[rows_half(right, 1)])
        rdma_to(out_ref.at[rows_half(right, 1)],
                out_ref.at[rows_half(right, 1)],
                bc_send_sems.at[4], bc_recv_sems.at[5], left)
        wait_recv_on(bc_recv_sems.at[2], out_ref.at[rows_half(left, 0)])
        rdma_to(out_ref.at[rows_half(left, 0)],
                out_ref.at[rows_half(left, 0)],
                bc_send_sems.at[5], bc_recv_sems.at[4], right)

        for s, ref_rows in ((0, rows_half(right, 0)), (3, rows_half(left, 1)),
                            (4, rows_half(c2, 0)), (5, rows_half(c2, 1))):
            wait_recv_on(bc_recv_sems.at[s], out_ref.at[ref_rows])

        for r in sends:
            r.wait_send()

    return pl.pallas_call(
        body,
        out_shape=jax.ShapeDtypeStruct((m, n), jnp.bfloat16),
        in_specs=[
            pl.BlockSpec(memory_space=pltpu.VMEM),
            pl.BlockSpec(memory_space=pltpu.VMEM),
            pl.BlockSpec(memory_space=pltpu.VMEM),
        ],
        out_specs=pl.BlockSpec(memory_space=pltpu.VMEM),
        scratch_shapes=[
            pltpu.VMEM((N_DEV - 1, mc, n), jnp.bfloat16),
            pltpu.VMEM((4, hr, n), jnp.bfloat16),
            pltpu.VMEM((2, hr, n), jnp.bfloat16),
            pltpu.VMEM((2, hr, n), jnp.bfloat16),
            pltpu.VMEM(x.shape, jnp.bfloat16),
            pltpu.VMEM(W1.shape, jnp.bfloat16),
            pltpu.VMEM(W2.shape, jnp.bfloat16),
            pltpu.SemaphoreType.DMA((6,)),
            pltpu.SemaphoreType.DMA((6,)),
            pltpu.SemaphoreType.DMA((6,)),
            pltpu.SemaphoreType.DMA((6,)),
        ],
        compiler_params=pltpu.CompilerParams(collective_id=0),
    )(x, W1, W2)
